# baseline (speedup 1.0000x reference)
import functools

import jax
import jax.numpy as jnp
from jax import lax
from jax.experimental import pallas as pl
from jax.experimental.pallas import tpu as pltpu
from jax.experimental.pallas import tpu_sc as plsc

S = 16384
D = 128
NC = 2
NS = 16
NW = NC * NS
SEG = S // NW
CH = 128
NCHUNK = SEG // CH
L = 16


def _take(v, idx):
    return v.at[idx].get(mode="promise_in_bounds")


_SHIFTS = (1, 2, 4, 8)
_LOG2 = {1: 0, 2: 1, 4: 2, 8: 3}


def _cumsum16(v, lanes):
    for k in _SHIFTS:
        w = jnp.minimum(lax.shift_right_logical(lanes, _LOG2[k]), 1)
        v = v + _take(v, jnp.maximum(lanes - k, 0)) * w
    return v


def _allsum16(v, lanes):
    for k in _SHIFTS:
        v = v + _take(v, lanes ^ k)
    return v


@functools.partial(
    pl.kernel,
    mesh=plsc.VectorSubcoreMesh(core_axis_name="c", subcore_axis_name="s"),
    out_type=jax.ShapeDtypeStruct((S, D), jnp.float32),
    scratch_types=[
        pltpu.VMEM((S,), jnp.int32),
        pltpu.VMEM((NCHUNK, CH), jnp.int32),
        pltpu.VMEM((NCHUNK, CH, D), jnp.float32),
        pltpu.SemaphoreType.DMA,
        pltpu.SemaphoreType.DMA,
        pltpu.SemaphoreType.DMA,
        pltpu.SemaphoreType.DMA,
        pltpu.SemaphoreType.DMA,
    ],
)
def _gather_kernel(mask_hbm, src_hbm, out_hbm,
                   mask_v, idx_v, rows_v, sem_g0, sem_g1, sem_g2, sem_g3, sem_o):
    c = lax.axis_index("c")
    s = lax.axis_index("s")
    wid = c * NS + s
    lanes = lax.iota(jnp.int32, L)
    last = lanes * 0 + (L - 1)

    pltpu.sync_copy(mask_hbm, mask_v)

    zeros = jnp.zeros((L,), jnp.int32)

    def pre_body(i, accs):
        a0, a1, a2, a3 = accs
        b = i * (8 * L)
        for j in range(0, 8, 4):
            a0 = a0 + mask_v[pl.ds(b + j * L, L)]
            a1 = a1 + mask_v[pl.ds(b + (j + 1) * L, L)]
            a2 = a2 + mask_v[pl.ds(b + (j + 2) * L, L)]
            a3 = a3 + mask_v[pl.ds(b + (j + 3) * L, L)]
        return a0, a1, a2, a3
    a0, a1, a2, a3 = lax.fori_loop(
        0, wid * (SEG // (8 * L)), pre_body, (zeros, zeros, zeros, zeros))
    off = _allsum16((a0 + a1) + (a2 + a3), lanes)

    base = wid * SEG

    def cs_body(j, carry):
        v = mask_v[pl.ds(base + j * L, L)]
        cs = _cumsum16(v, lanes) + carry
        idx = jnp.maximum(cs - 1, 0)
        idx_v[j >> 3, pl.ds((j & 7) * L, L)] = idx
        return _take(cs, last)

    sems = (sem_g0, sem_g1, sem_g2, sem_g3)
    carry = off
    gathers = []
    for j in range(NCHUNK):
        carry = lax.fori_loop(j * (CH // L), (j + 1) * (CH // L), cs_body, carry)
        gathers.append(
            pltpu.async_copy(src_hbm.at[idx_v.at[j]], rows_v.at[j], sems[j]))
    outs = []
    for j in range(NCHUNK):
        gathers[j].wait()
        outs.append(pltpu.async_copy(
            rows_v.at[j], out_hbm.at[pl.ds(base + j * CH, CH)], sem_o))
    for cp in outs:
        cp.wait()


def kernel(mask_1d, source):
    return _gather_kernel(mask_1d.astype(jnp.int32), source)


# speedup vs baseline: 1.0272x; 1.0272x over previous
import functools

import jax
import jax.numpy as jnp
from jax import lax
from jax.experimental import pallas as pl
from jax.experimental.pallas import tpu as pltpu
from jax.experimental.pallas import tpu_sc as plsc

S = 16384
D = 128
NC = 2
NS = 16
NW = NC * NS
SEG = S // NW
CH = 128
NCHUNK = SEG // CH
L = 16


def _take(v, idx):
    return v.at[idx].get(mode="promise_in_bounds")


_SHIFTS = (1, 2, 4, 8)
_LOG2 = {1: 0, 2: 1, 4: 2, 8: 3}


def _cumsum16(v, lanes):
    for k in _SHIFTS:
        w = jnp.minimum(lax.shift_right_logical(lanes, _LOG2[k]), 1)
        v = v + _take(v, jnp.maximum(lanes - k, 0)) * w
    return v


def _allsum16(v, lanes):
    for k in _SHIFTS:
        v = v + _take(v, lanes ^ k)
    return v


@functools.partial(
    pl.kernel,
    mesh=plsc.VectorSubcoreMesh(core_axis_name="c", subcore_axis_name="s"),
    out_type=jax.ShapeDtypeStruct((S, D), jnp.float32),
    scratch_types=[
        pltpu.VMEM((S,), jnp.int32),
        pltpu.VMEM((NCHUNK, CH), jnp.int32),
        pltpu.VMEM((NCHUNK, CH, D), jnp.float32),
        pltpu.SemaphoreType.DMA,
        pltpu.SemaphoreType.DMA,
        pltpu.SemaphoreType.DMA,
        pltpu.SemaphoreType.DMA,
        pltpu.SemaphoreType.DMA,
    ],
)
def _gather_kernel(mask_hbm, src_hbm, out_hbm,
                   mask_v, idx_v, rows_v, sem_g0, sem_g1, sem_g2, sem_g3, sem_o):
    c = lax.axis_index("c")
    s = lax.axis_index("s")
    wid = c * NS + s
    lanes = lax.iota(jnp.int32, L)
    last = lanes * 0 + (L - 1)

    pltpu.sync_copy(mask_hbm, mask_v)

    zeros = jnp.zeros((L,), jnp.int32)

    def pre_body(i, accs):
        a0, a1, a2, a3 = accs
        b = i * (16 * L)
        for j in range(0, 16, 4):
            a0 = a0 + mask_v[pl.ds(b + j * L, L)]
            a1 = a1 + mask_v[pl.ds(b + (j + 1) * L, L)]
            a2 = a2 + mask_v[pl.ds(b + (j + 2) * L, L)]
            a3 = a3 + mask_v[pl.ds(b + (j + 3) * L, L)]
        return a0, a1, a2, a3
    a0, a1, a2, a3 = lax.fori_loop(
        0, wid * (SEG // (16 * L)), pre_body, (zeros, zeros, zeros, zeros))
    off = _allsum16((a0 + a1) + (a2 + a3), lanes)

    base = wid * SEG

    def cs_body(j, carry):
        v = mask_v[pl.ds(base + j * L, L)]
        cs = _cumsum16(v, lanes) + carry
        idx = jnp.maximum(cs - 1, 0)
        idx_v[j >> 3, pl.ds((j & 7) * L, L)] = idx
        return _take(cs, last)
    lax.fori_loop(0, SEG // L, cs_body, off)

    sems = (sem_g0, sem_g1, sem_g2, sem_g3)
    gathers = [
        pltpu.async_copy(src_hbm.at[idx_v.at[j]], rows_v.at[j], sems[j])
        for j in range(NCHUNK)
    ]
    outs = []
    for j in range(NCHUNK):
        gathers[j].wait()
        outs.append(pltpu.async_copy(
            rows_v.at[j], out_hbm.at[pl.ds(base + j * CH, CH)], sem_o))
    for cp in outs:
        cp.wait()


def kernel(mask_1d, source):
    return _gather_kernel(mask_1d.astype(jnp.int32), source)


# speedup vs baseline: 1.0659x; 1.0377x over previous
import functools

import jax
import jax.numpy as jnp
from jax import lax
from jax.experimental import pallas as pl
from jax.experimental.pallas import tpu as pltpu
from jax.experimental.pallas import tpu_sc as plsc

S = 16384
D = 128
NC = 2
NS = 16
NW = NC * NS
SEG = S // NW
CH = 128
NCHUNK = SEG // CH
L = 16


def _take(v, idx):
    return v.at[idx].get(mode="promise_in_bounds")


_SHIFTS = (1, 2, 4, 8)
_LOG2 = {1: 0, 2: 1, 4: 2, 8: 3}


def _cumsum16(v, lanes):
    for k in _SHIFTS:
        w = jnp.minimum(lax.shift_right_logical(lanes, _LOG2[k]), 1)
        v = v + _take(v, jnp.maximum(lanes - k, 0)) * w
    return v


def _allsum16(v, lanes):
    for k in _SHIFTS:
        v = v + _take(v, lanes ^ k)
    return v


@functools.partial(
    pl.kernel,
    mesh=plsc.VectorSubcoreMesh(core_axis_name="c", subcore_axis_name="s"),
    out_type=jax.ShapeDtypeStruct((S, D), jnp.float32),
    scratch_types=[
        pltpu.VMEM((2, SEG), jnp.int32),
        pltpu.VMEM((2, L), jnp.int32),
        pltpu.VMEM((NCHUNK, CH), jnp.int32),
        pltpu.VMEM((NCHUNK, CH, D), jnp.float32),
        pltpu.SMEM((1,), jnp.int32),
        pltpu.SemaphoreType.DMA,
        pltpu.SemaphoreType.DMA,
        pltpu.SemaphoreType.DMA,
        pltpu.SemaphoreType.DMA,
        pltpu.SemaphoreType.DMA,
    ],
)
def _gather_kernel(mask_hbm, src_hbm, out_hbm,
                   mask_v, tot_v, idx_v, rows_v, cnt_ref,
                   sem_g0, sem_g1, sem_g2, sem_g3, sem_o):
    c = lax.axis_index("c")
    s = lax.axis_index("s")
    wid = c * NS + s
    lanes = lax.iota(jnp.int32, L)
    last = lanes * 0 + (L - 1)
    zeros = jnp.zeros((L,), jnp.int32)

    cnt_ref[0] = 0
    plsc.subcore_barrier()

    first_seg = s
    second_seg = s + NS
    own_row = c
    pltpu.sync_copy(mask_hbm.at[pl.ds((s + c * NS) * SEG, SEG)], mask_v.at[c])
    pltpu.sync_copy(mask_hbm.at[pl.ds((s + (1 - c) * NS) * SEG, SEG)],
                    mask_v.at[1 - c])

    for h in range(2):
        acc = zeros
        for j in range(0, SEG // L, 4):
            acc = (acc + mask_v[h, pl.ds(j * L, L)]
                   + mask_v[h, pl.ds((j + 1) * L, L)]
                   + mask_v[h, pl.ds((j + 2) * L, L)]
                   + mask_v[h, pl.ds((j + 3) * L, L)])
        tot_v[h, :] = _allsum16(acc, lanes)
    t_first = tot_v[0, :][0]
    t_second = tot_v[1, :][0]

    for t in range(NS):
        T = c * NS + t
        m1 = (first_seg < T).astype(jnp.int32)
        m2 = (second_seg < T).astype(jnp.int32)
        contrib = m1 * t_first + m2 * t_second
        plsc.fetch_and_add(cnt_ref.at[0], contrib, subcore_id=t)
    plsc.subcore_barrier()
    off = zeros + cnt_ref[0]

    base = wid * SEG

    def cs_body(j, carry):
        v = mask_v[own_row, pl.ds(j * L, L)]
        cs = _cumsum16(v, lanes) + carry
        idx = jnp.maximum(cs - 1, 0)
        idx_v[j >> 3, pl.ds((j & 7) * L, L)] = idx
        return _take(cs, last)
    lax.fori_loop(0, SEG // L, cs_body, off)

    sems = (sem_g0, sem_g1, sem_g2, sem_g3)
    gathers = [
        pltpu.async_copy(src_hbm.at[idx_v.at[j]], rows_v.at[j], sems[j])
        for j in range(NCHUNK)
    ]
    outs = []
    for j in range(NCHUNK):
        gathers[j].wait()
        outs.append(pltpu.async_copy(
            rows_v.at[j], out_hbm.at[pl.ds(base + j * CH, CH)], sem_o))
    for cp in outs:
        cp.wait()


def kernel(mask_1d, source):
    return _gather_kernel(mask_1d.astype(jnp.int32), source)


# speedup vs baseline: 1.0845x; 1.0175x over previous
import functools

import jax
import jax.numpy as jnp
from jax import lax
from jax.experimental import pallas as pl
from jax.experimental.pallas import tpu as pltpu
from jax.experimental.pallas import tpu_sc as plsc

S = 16384
D = 128
NC = 2
NS = 16
NW = NC * NS
SEG = S // NW
CH = 128
NCHUNK = SEG // CH
L = 16


def _take(v, idx):
    return v.at[idx].get(mode="promise_in_bounds")


_SHIFTS = (1, 2, 4, 8)
_LOG2 = {1: 0, 2: 1, 4: 2, 8: 3}


def _cumsum16(v, lanes):
    for k in _SHIFTS:
        w = jnp.minimum(lax.shift_right_logical(lanes, _LOG2[k]), 1)
        v = v + _take(v, jnp.maximum(lanes - k, 0)) * w
    return v


def _allsum16(v, lanes):
    for k in _SHIFTS:
        v = v + _take(v, lanes ^ k)
    return v


@functools.partial(
    pl.kernel,
    mesh=plsc.VectorSubcoreMesh(core_axis_name="c", subcore_axis_name="s"),
    out_type=jax.ShapeDtypeStruct((S, D), jnp.float32),
    scratch_types=[
        pltpu.VMEM((2, SEG), jnp.int32),
        pltpu.VMEM((2, L), jnp.int32),
        pltpu.VMEM((NCHUNK, CH), jnp.int32),
        pltpu.VMEM((NCHUNK, CH, D), jnp.float32),
        pltpu.SMEM((1,), jnp.int32),
        pltpu.SemaphoreType.DMA,
        pltpu.SemaphoreType.DMA,
        pltpu.SemaphoreType.DMA,
        pltpu.SemaphoreType.DMA,
        pltpu.SemaphoreType.DMA,
    ],
)
def _gather_kernel(mask_hbm, src_hbm, out_hbm,
                   mask_v, tot_v, idx_v, rows_v, cnt_ref,
                   sem_g0, sem_g1, sem_g2, sem_g3, sem_o):
    c = lax.axis_index("c")
    s = lax.axis_index("s")
    wid = c * NS + s
    lanes = lax.iota(jnp.int32, L)
    last = lanes * 0 + (L - 1)
    zeros = jnp.zeros((L,), jnp.int32)

    first_seg = s
    second_seg = s + NS
    own_row = c
    ld0 = pltpu.async_copy(mask_hbm.at[pl.ds((s + c * NS) * SEG, SEG)],
                           mask_v.at[c], sem_g0)
    ld1 = pltpu.async_copy(mask_hbm.at[pl.ds((s + (1 - c) * NS) * SEG, SEG)],
                           mask_v.at[1 - c], sem_g1)
    cnt_ref[0] = 0
    plsc.subcore_barrier()
    ld0.wait()
    ld1.wait()

    for h in range(2):
        acc = zeros
        for j in range(0, SEG // L, 4):
            acc = (acc + mask_v[h, pl.ds(j * L, L)]
                   + mask_v[h, pl.ds((j + 1) * L, L)]
                   + mask_v[h, pl.ds((j + 2) * L, L)]
                   + mask_v[h, pl.ds((j + 3) * L, L)])
        tot_v[h, :] = _allsum16(acc, lanes)
    t_first = tot_v[0, :][0]
    t_second = tot_v[1, :][0]

    for t in range(NS):
        T = c * NS + t
        m1 = (first_seg < T).astype(jnp.int32)
        m2 = (second_seg < T).astype(jnp.int32)
        contrib = m1 * t_first + m2 * t_second
        plsc.fetch_and_add(cnt_ref.at[0], contrib, subcore_id=t)
    plsc.subcore_barrier()
    off = zeros + cnt_ref[0]

    base = wid * SEG

    def cs_body(j, carry):
        v = mask_v[own_row, pl.ds(j * L, L)]
        cs = _cumsum16(v, lanes) + carry
        idx = jnp.maximum(cs - 1, 0)
        idx_v[j >> 3, pl.ds((j & 7) * L, L)] = idx
        return _take(cs, last)
    lax.fori_loop(0, SEG // L, cs_body, off)

    sems = (sem_g0, sem_g1, sem_g2, sem_g3)
    gathers = [
        pltpu.async_copy(src_hbm.at[idx_v.at[j]], rows_v.at[j], sems[j])
        for j in range(NCHUNK)
    ]
    outs = []
    for j in range(NCHUNK):
        gathers[j].wait()
        outs.append(pltpu.async_copy(
            rows_v.at[j], out_hbm.at[pl.ds(base + j * CH, CH)], sem_o))
    for cp in outs:
        cp.wait()


def kernel(mask_1d, source):
    return _gather_kernel(mask_1d.astype(jnp.int32), source)
